# Initial kernel scaffold; baseline (speedup 1.0000x reference)
import jax
import jax.numpy as jnp
from jax.experimental import pallas as pl


def kernel(actions, table):
    raise NotImplementedError("write your pallas kernel here")


# speedup vs baseline: 3.9557x; 3.9557x over previous
import functools

import jax
import jax.numpy as jnp
from jax import lax
from jax.experimental import pallas as pl
from jax.experimental.pallas import tpu as pltpu
from jax.experimental.pallas import tpu_sc as plsc

_D = 64
_B_TOTAL = 4096 * 200

_info = plsc.get_sparse_core_info()
_NC, _NS = _info.num_cores, _info.num_subcores
_NW = _NC * _NS
_B_PER_W = _B_TOTAL // _NW
_CHUNK = 512
_NCHUNKS = _B_PER_W // _CHUNK


def _embed_body(idx_hbm, table_hbm, out_hbm, idx_v, rows_v, gsem):
    wid = lax.axis_index("s") * _NC + lax.axis_index("c")
    base = wid * _B_PER_W

    def chunk(g, carry):
        pltpu.sync_copy(idx_hbm.at[wid, g], idx_v)
        pltpu.async_copy(table_hbm.at[idx_v], rows_v, gsem).wait()
        pltpu.sync_copy(rows_v, out_hbm.at[pl.ds(base + g * _CHUNK, _CHUNK)])
        return carry

    lax.fori_loop(0, _NCHUNKS, chunk, 0)


_mesh = plsc.VectorSubcoreMesh(core_axis_name="c", subcore_axis_name="s")

_embed = functools.partial(
    pl.kernel,
    mesh=_mesh,
    out_type=jax.ShapeDtypeStruct((_B_TOTAL, _D), jnp.float32),
    scratch_types=[
        pltpu.VMEM((_CHUNK,), jnp.int32),
        pltpu.VMEM((_CHUNK, _D), jnp.float32),
        pltpu.SemaphoreType.DMA,
    ],
    compiler_params=pltpu.CompilerParams(use_tc_tiling_on_sc=False),
)(_embed_body)


@jax.jit
def kernel(actions, table):
    idx = actions.reshape(_NW, _NCHUNKS, _CHUNK).astype(jnp.int32)
    out = _embed(idx, table)
    return out.reshape(actions.shape[0], actions.shape[1], _D)


# speedup vs baseline: 4.2745x; 1.0806x over previous
import functools

import jax
import jax.numpy as jnp
from jax import lax
from jax.experimental import pallas as pl
from jax.experimental.pallas import tpu as pltpu
from jax.experimental.pallas import tpu_sc as plsc

_D = 64
_B_TOTAL = 4096 * 200

_info = plsc.get_sparse_core_info()
_NC, _NS = _info.num_cores, _info.num_subcores
_NW = _NC * _NS
_B_PER_W = _B_TOTAL // _NW
_CHUNK = 256
_NCHUNKS = _B_PER_W // _CHUNK
_NBUF = 4
_DI = 4
_DG = 2


def _embed_body(idx_hbm, table_hbm, out_hbm, idx_v, rows_v, isem, gsem, osem):
    wid = lax.axis_index("s") * _NC + lax.axis_index("c")
    base = wid * _B_PER_W

    def idx_copy(g, slot):
        return pltpu.make_async_copy(idx_hbm.at[wid, g], idx_v.at[slot],
                                     isem.at[slot])

    def gather(slot):
        return pltpu.make_async_copy(table_hbm.at[idx_v.at[slot]],
                                     rows_v.at[slot], gsem.at[slot])

    def out_copy(g, slot):
        return pltpu.make_async_copy(
            rows_v.at[slot], out_hbm.at[pl.ds(base + g * _CHUNK, _CHUNK)],
            osem.at[slot])

    for g in range(_DI):
        idx_copy(g, g % _NBUF).start()
    for g in range(_DG):
        idx_copy(g, g % _NBUF).wait()
        gather(g % _NBUF).start()

    def step(i, carry):
        g0 = i * _NBUF
        for j in range(_NBUF):
            g = g0 + j
            gather(j).wait()
            out_copy(g, j).start()
            gi = g + _DI

            @pl.when(gi < _NCHUNKS)
            def _():
                idx_copy(gi, j).start()

            gg = g + _DG
            gslot = (j + _DG) % _NBUF

            @pl.when(gg < _NCHUNKS)
            def _():
                @pl.when(gg >= _NBUF)
                def _():
                    out_copy(0, gslot).wait()
                idx_copy(0, gslot).wait()
                gather(gslot).start()

        return carry

    lax.fori_loop(0, _NCHUNKS // _NBUF, step, 0)

    for j in range(_NBUF):
        out_copy(0, j).wait()


_mesh = plsc.VectorSubcoreMesh(core_axis_name="c", subcore_axis_name="s")

_embed = functools.partial(
    pl.kernel,
    mesh=_mesh,
    out_type=jax.ShapeDtypeStruct((_B_TOTAL, _D), jnp.float32),
    scratch_types=[
        pltpu.VMEM((_NBUF, _CHUNK), jnp.int32),
        pltpu.VMEM((_NBUF, _CHUNK, _D), jnp.float32),
        pltpu.SemaphoreType.DMA((_NBUF,)),
        pltpu.SemaphoreType.DMA((_NBUF,)),
        pltpu.SemaphoreType.DMA((_NBUF,)),
    ],
    compiler_params=pltpu.CompilerParams(use_tc_tiling_on_sc=False),
)(_embed_body)


@jax.jit
def kernel(actions, table):
    idx = actions.reshape(_NW, _NCHUNKS, _CHUNK).astype(jnp.int32)
    out = _embed(idx, table)
    return out.reshape(actions.shape[0], actions.shape[1], _D)
